# baseline (speedup 1.0000x reference)
import jax
import jax.numpy as jnp
from jax import lax
from jax.experimental import pallas as pl
from jax.experimental.pallas import tpu as pltpu
from jax.experimental.pallas import tpu_sc as plsc

NLEV = 16
NFEAT = 2
HSIZE = 524288
BATCH = 524288
RES = [16, 22, 30, 42, 58, 80, 110, 152, 210, 290, 400, 552, 762, 1052, 1452, 2048]
P1 = -1640531535
P2 = 805459861
MASK = HSIZE - 1

NC, NS = 2, 16
NW = NC * NS
NPW = BATCH // NW
C = 1024
NCH = NPW // C
IPC = C * NLEV * NFEAT
GROWS = 1024
NG = IPC // GROWS


def _sc_body(coords_hbm, tflat_hbm, out_hbm, cbuf, ibuf, obuf, sem):
    wid = lax.axis_index("s") * jnp.int32(NC) + lax.axis_index("c")
    wbase = wid * jnp.int32(NPW)
    lanes = lax.iota(jnp.int32, 16)
    l32 = lanes * jnp.int32(2 * NLEV)

    def chunk_body(k, carry):
        base = wbase + k * jnp.int32(C)
        pltpu.sync_copy(coords_hbm.at[:, pl.ds(base, C)], cbuf)

        def hash_step(s, carry2):
            col = s * jnp.int32(16)
            x = cbuf[0, pl.ds(col, 16)]
            y = cbuf[1, pl.ds(col, 16)]
            z = cbuf[2, pl.ds(col, 16)]
            pb = l32 + s * jnp.int32(16 * 2 * NLEV)
            for lvl in range(NLEV):
                r = jnp.float32(RES[lvl])
                gx = (x * r).astype(jnp.int32)
                gy = (y * r).astype(jnp.int32)
                gz = (z * r).astype(jnp.int32)
                h = gx ^ (gy * jnp.int32(P1)) ^ (gz * jnp.int32(P2))
                e0 = (((h & jnp.int32(MASK)) | jnp.int32(lvl << 19))
                      * jnp.int32(2))
                plsc.store_scatter(ibuf, [pb + jnp.int32(2 * lvl)], e0)
                plsc.store_scatter(ibuf, [pb + jnp.int32(2 * lvl + 1)],
                                   e0 + jnp.int32(1))
            return carry2

        lax.fori_loop(jnp.int32(0), jnp.int32(C // 16), hash_step,
                      jnp.int32(0))

        def g_fire(j, carry2):
            pltpu.async_copy(
                tflat_hbm.at[ibuf.at[pl.ds(j * jnp.int32(GROWS), GROWS)]],
                obuf.at[pl.ds(j * jnp.int32(GROWS), GROWS)],
                sem)
            return carry2

        def g_drain(j, carry2):
            pltpu.make_async_copy(
                tflat_hbm.at[ibuf.at[pl.ds(j * jnp.int32(GROWS), GROWS)]],
                obuf.at[pl.ds(j * jnp.int32(GROWS), GROWS)],
                sem).wait()
            return carry2

        lax.fori_loop(jnp.int32(0), jnp.int32(NG), g_fire, jnp.int32(0))
        lax.fori_loop(jnp.int32(0), jnp.int32(NG), g_drain, jnp.int32(0))

        obase = base * jnp.int32(NLEV * NFEAT)
        pltpu.sync_copy(obuf, out_hbm.at[pl.ds(obase, IPC)])
        return carry

    lax.fori_loop(jnp.int32(0), jnp.int32(NCH), chunk_body, jnp.int32(0))


def kernel(coords, tables):
    coords_t = coords.T.astype(jnp.float32)
    tflat = tables.reshape(NLEV * HSIZE * NFEAT)
    mesh = plsc.VectorSubcoreMesh(core_axis_name="c", subcore_axis_name="s")
    f = pl.kernel(
        _sc_body,
        mesh=mesh,
        compiler_params=pltpu.CompilerParams(
            use_tc_tiling_on_sc=False, needs_layout_passes=False),
        out_type=jax.ShapeDtypeStruct((BATCH * NLEV * NFEAT,), jnp.float32),
        scratch_types=[
            pltpu.VMEM((3, C), jnp.float32),
            pltpu.VMEM((IPC,), jnp.int32),
            pltpu.VMEM((IPC,), jnp.float32),
            pltpu.SemaphoreType.DMA,
        ],
    )
    return f(coords_t, tflat).reshape(BATCH, NLEV * NFEAT)
